# baseline (speedup 1.0000x reference)
import functools

import jax
import jax.numpy as jnp
from jax import lax
from jax.experimental import pallas as pl
from jax.experimental.pallas import tpu as pltpu
from jax.experimental.pallas import tpu_sc as plsc

N = 10000
E = 320000
IN_CH = 128
HID = 64
HEADS = 4
OUT_CH = 128
HH = HEADS * HID
HALF = HH // 2

NSUB = 16
N_PAD = 10112
NPS = N_PAD // NSUB
E_PAD = 327680
EPS = E_PAD // NSUB
CHG = 40
NCHG = EPS // CHG
SBG = 32
NSBG = NCHG // SBG
CHS = 80
NCHS = EPS // CHS
SBS = 32
NSBS = NCHS // SBS
HBW = 144


def _tc1_body(x_ref, wgT_ref, ab_ref, hb_ref, adst_ref, exl_ref):
    x = x_ref[...]
    nb = x.shape[0]
    h = jnp.dot(x, wgT_ref[...], preferred_element_type=jnp.float32)
    a8 = jnp.dot(h, ab_ref[...], preferred_element_type=jnp.float32)
    z = jnp.zeros((nb, 12), jnp.float32)
    asrc16 = jnp.concatenate([a8[:, 0:4], z], axis=1)
    for c in range(2):
        hc = h[:, c * HALF:(c + 1) * HALF]
        hb_ref[c] = jnp.concatenate([hc, asrc16], axis=1)
    adst_ref[...] = jnp.concatenate([a8[:, 4:8], z], axis=1)
    al = a8[:, 0:4] + a8[:, 4:8]
    al = jnp.where(al > 0.0, al, 0.2 * al)
    exl_ref[...] = jnp.exp(al)


NPAIR = SBG // 2
NQUAD = SBS // 4


def _gat_sc_body(src2_h, dst2_h, adst_h, hb_h, z128_h, z16_h,
                 msg_h, den_h,
                 sball, dball, adg, exb, hrows, msgb, msg_sp, den_sp,
                 gsem0, gsem1, ssem0, ssem1):
    c = lax.axis_index("c")
    s = lax.axis_index("s")
    nbase = s * NPS
    pltpu.sync_copy(z128_h.at[pl.ds(nbase, NPS)], msg_sp.at[pl.ds(nbase, NPS)])
    pltpu.sync_copy(z16_h.at[pl.ds(nbase, NPS)], den_sp.at[pl.ds(nbase, NPS)])
    plsc.subcore_barrier()

    def run_half(h_half, msg_half, col0, do_den):
        gs = (gsem0, gsem1)
        ss = (ssem0, ssem1)

        def issue_gather(k, b):
            pltpu.async_copy(adst_h.at[dball.at[k]], adg.at[b], gs[b])
            pltpu.async_copy(h_half.at[sball.at[k]], hrows.at[b], gs[b])

        def wait_gather(k, b):
            pltpu.make_async_copy(adst_h.at[dball.at[k]], adg.at[b], gs[b]).wait()
            pltpu.make_async_copy(h_half.at[sball.at[k]], hrows.at[b], gs[b]).wait()

        def drain_scatter(k, b):
            pltpu.make_async_copy(msgb.at[b], msg_sp.at[dball.at[k]], ss[b]).wait()
            if do_den:
                pltpu.make_async_copy(exb.at[b], den_sp.at[dball.at[k]], ss[b]).wait()

        def compute_and_scatter(k, b):
            @plsc.parallel_loop(0, CHG, step=1, unroll=4)
            def ebody(j):
                asr = hrows[b, j, pl.ds(HALF, 16)]
                a = asr + adg[b, j, :]
                a = jnp.maximum(a, a * 0.2)
                e = jnp.exp(a)
                exb[b, j, :] = e
                e0 = e[col0]
                e1 = e[col0 + 1]
                for q in range(8):
                    ehq = e0 if q < 4 else e1
                    msgb[b, j, pl.ds(q * 16, 16)] = hrows[b, j, pl.ds(q * 16, 16)] * ehq
            pltpu.async_copy(msgb.at[b], msg_sp.at[dball.at[k]], ss[b], add=True)
            if do_den:
                pltpu.async_copy(exb.at[b], den_sp.at[dball.at[k]], ss[b], add=True)

        def super_t(t, carry):
            tb = s * NCHG + t * SBG
            pltpu.sync_copy(src2_h.at[pl.ds(tb, SBG)], sball)
            pltpu.sync_copy(dst2_h.at[pl.ds(tb, SBG)], dball)
            issue_gather(0, 0)

            def pairbody(i, carry2):
                k0 = 2 * i
                wait_gather(k0, 0)

                @pl.when(i > 0)
                def _():
                    drain_scatter(k0 - 1, 1)

                issue_gather(k0 + 1, 1)
                compute_and_scatter(k0, 0)

                wait_gather(k0 + 1, 1)
                drain_scatter(k0, 0)

                @pl.when(i < NPAIR - 1)
                def _():
                    issue_gather(k0 + 2, 0)

                compute_and_scatter(k0 + 1, 1)
                return carry2

            lax.fori_loop(0, NPAIR, pairbody, 0)
            drain_scatter(SBG - 1, 1)
            return carry

        lax.fori_loop(0, NSBG, super_t, 0)
        plsc.subcore_barrier()
        pltpu.sync_copy(msg_sp.at[pl.ds(nbase, NPS)], msg_half.at[pl.ds(nbase, NPS)])
        if do_den:
            pltpu.sync_copy(den_sp.at[pl.ds(nbase, NPS)], den_h.at[pl.ds(nbase, NPS)])

    @pl.when(c == 0)
    def _():
        run_half(hb_h.at[0], msg_h.at[0], 0, True)

    @pl.when(c == 1)
    def _():
        run_half(hb_h.at[1], msg_h.at[1], 2, False)


def _sage_sc_body(src2_h, dst2_h, h2_h, z128_h, agg_h,
                  sball, dball, hrows,
                  agg_sp,
                  gsem0, gsem1, gsem2, gsem3, ssem0, ssem1, ssem2, ssem3):
    c = lax.axis_index("c")
    s = lax.axis_index("s")
    nbase = s * NPS
    pltpu.sync_copy(z128_h.at[pl.ds(nbase, NPS)], agg_sp.at[pl.ds(nbase, NPS)])
    plsc.subcore_barrier()

    def run_half(h_half, out_half):
        gs = (gsem0, gsem1, gsem2, gsem3)
        ss = (ssem0, ssem1, ssem2, ssem3)

        def issue_gather(k, b):
            pltpu.async_copy(h_half.at[sball.at[k]], hrows.at[b], gs[b])

        def wait_gather(k, b):
            pltpu.make_async_copy(h_half.at[sball.at[k]], hrows.at[b], gs[b]).wait()

        def drain_scatter(k, b):
            pltpu.make_async_copy(hrows.at[b], agg_sp.at[dball.at[k]], ss[b]).wait()

        def super_t(t, carry):
            tb = s * NCHS + t * SBS
            pltpu.sync_copy(src2_h.at[pl.ds(tb, SBS)], sball)
            pltpu.sync_copy(dst2_h.at[pl.ds(tb, SBS)], dball)
            for b in range(3):
                issue_gather(b, b)

            def quadbody(i, carry2):
                for b in range(4):
                    k = 4 * i + b
                    bn = (b + 3) % 4
                    wait_gather(k, b)
                    if b == 0:
                        @pl.when(i > 0)
                        def _():
                            drain_scatter(k - 1, bn)

                        issue_gather(k + 3, bn)
                    else:
                        @pl.when(i < NQUAD - 1)
                        def _():
                            drain_scatter(k - 1, bn)
                            issue_gather(k + 3, bn)
                    pltpu.async_copy(hrows.at[b], agg_sp.at[dball.at[k]], ss[b], add=True)
                return carry2

            lax.fori_loop(0, NQUAD, quadbody, 0)
            for b in range(4):
                drain_scatter(SBS - 4 + b, b)
            return carry

        lax.fori_loop(0, NSBS, super_t, 0)
        plsc.subcore_barrier()
        pltpu.sync_copy(agg_sp.at[pl.ds(nbase, NPS)], out_half.at[pl.ds(nbase, NPS)])

    @pl.when(c == 0)
    def _():
        run_half(h2_h.at[0], agg_h.at[0])

    @pl.when(c == 1)
    def _():
        run_half(h2_h.at[1], agg_h.at[1])


def _tc2_body(msg_ref, h_ref, den_ref, exl_ref, S_ref, bg_ref, bng_ref, bnb_ref,
              h2_ref):
    exl = exl_ref[0:N, :]
    S = S_ref[0]
    elb = jnp.dot(exl, S, preferred_element_type=jnp.float32)
    r4 = 1.0 / (den_ref[0:N, 0:4] + exl + 1e-16)
    rb = jnp.dot(r4, S, preferred_element_type=jnp.float32)
    gat = (msg_ref[0, 0:N, :] + elb * h_ref[0, 0:N, :]) * rb + bg_ref[0]
    mu = jnp.mean(gat, axis=0, keepdims=True)
    xc = gat - mu
    var = jnp.mean(xc * xc, axis=0, keepdims=True)
    xn = xc * lax.rsqrt(var + 1e-5) * bng_ref[0] + bnb_ref[0]
    h2 = jnp.where(xn > 0.0, xn, jnp.exp(jnp.minimum(xn, 0.0)) - 1.0)
    h2_ref[0, 0:N, :] = h2
    h2_ref[0, N:N_PAD, :] = jnp.zeros((N_PAD - N, HALF), jnp.float32)


def _tc3_body(agg_ref, h2_ref, den_ref, x_ref, wsl_ref, wsr_ref, wsk_ref,
              bt_ref, out_ref):
    inv = 1.0 / jnp.maximum(den_ref[:, 4:5], 1.0)
    m0 = agg_ref[0] * inv
    m1 = agg_ref[1] * inv
    out = jnp.dot(m0, wsl_ref[0:HALF, :], preferred_element_type=jnp.float32)
    out = out + jnp.dot(m1, wsl_ref[HALF:HH, :], preferred_element_type=jnp.float32)
    out = out + jnp.dot(h2_ref[0], wsr_ref[0:HALF, :], preferred_element_type=jnp.float32)
    out = out + jnp.dot(h2_ref[1], wsr_ref[HALF:HH, :], preferred_element_type=jnp.float32)
    out = out + jnp.dot(x_ref[...], wsk_ref[...], preferred_element_type=jnp.float32)
    out_ref[...] = out + bt_ref[...]


def kernel(x, edge_index, W_gat, att_src, att_dst, b_gat, bn_g, bn_b,
           W_sl, W_sr, b_s, W_skip, b_skip):
    src = edge_index[0]
    dst = edge_index[1]
    pad_idx = (jnp.arange(E_PAD - E, dtype=jnp.int32) % 16) + N
    src_p = jnp.concatenate([src, pad_idx])
    dst_p = jnp.concatenate([dst, pad_idx])
    x_p = jnp.pad(x, ((0, N_PAD - N), (0, 0)))

    eye = jnp.eye(HEADS, dtype=jnp.float32)
    A_src = (att_src[:, :, None] * eye[:, None, :]).reshape(HH, HEADS)
    A_dst = (att_dst[:, :, None] * eye[:, None, :]).reshape(HH, HEADS)
    AB = jnp.concatenate([A_src, A_dst], axis=1)

    fhead = jnp.arange(HALF, dtype=jnp.int32) // HID
    g = jnp.arange(HEADS, dtype=jnp.int32)
    S = (g[None, :, None] == (2 * jnp.arange(2, dtype=jnp.int32)[:, None, None]
                              + fhead[None, None, :])).astype(jnp.float32)

    z128 = jnp.zeros((N_PAD, HALF), jnp.float32)
    z16 = jnp.zeros((N_PAD, 16), jnp.float32)

    B1 = N_PAD // 4
    hb, adst_p, exl = pl.pallas_call(
        _tc1_body,
        grid=(4,),
        in_specs=[
            pl.BlockSpec((B1, IN_CH), lambda i: (i, 0)),
            pl.BlockSpec((IN_CH, HH), lambda i: (0, 0)),
            pl.BlockSpec((HH, 8), lambda i: (0, 0)),
        ],
        out_specs=[
            pl.BlockSpec((2, B1, HBW), lambda i: (0, i, 0)),
            pl.BlockSpec((B1, 16), lambda i: (i, 0)),
            pl.BlockSpec((B1, 4), lambda i: (i, 0)),
        ],
        out_shape=[
            jax.ShapeDtypeStruct((2, N_PAD, HBW), jnp.float32),
            jax.ShapeDtypeStruct((N_PAD, 16), jnp.float32),
            jax.ShapeDtypeStruct((N_PAD, 4), jnp.float32),
        ],
    )(x_p, W_gat.T, AB)

    mesh = plsc.VectorSubcoreMesh(core_axis_name="c", subcore_axis_name="s")

    sc_params = pltpu.CompilerParams(use_tc_tiling_on_sc=False)

    gat_edge = pl.kernel(
        _gat_sc_body,
        compiler_params=sc_params,
        out_type=[
            jax.ShapeDtypeStruct((2, N_PAD, HALF), jnp.float32),
            jax.ShapeDtypeStruct((N_PAD, 16), jnp.float32),
        ],
        mesh=mesh,
        scratch_types=[
            pltpu.VMEM((SBG, CHG), jnp.int32),
            pltpu.VMEM((SBG, CHG), jnp.int32),
            pltpu.VMEM((2, CHG, 16), jnp.float32),
            pltpu.VMEM((2, CHG, 16), jnp.float32),
            pltpu.VMEM((2, CHG, HBW), jnp.float32),
            pltpu.VMEM((2, CHG, HALF), jnp.float32),
            pltpu.VMEM_SHARED((N_PAD, HALF), jnp.float32),
            pltpu.VMEM_SHARED((N_PAD, 16), jnp.float32),
            pltpu.SemaphoreType.DMA,
            pltpu.SemaphoreType.DMA,
            pltpu.SemaphoreType.DMA,
            pltpu.SemaphoreType.DMA,
        ],
    )
    src2g = src_p.reshape(NSUB * NCHG, CHG)
    dst2g = dst_p.reshape(NSUB * NCHG, CHG)
    msg, den16 = gat_edge(src2g, dst2g, adst_p, hb, z128, z16)

    h2 = pl.pallas_call(
        _tc2_body,
        grid=(2,),
        in_specs=[
            pl.BlockSpec((1, N_PAD, HALF), lambda c: (c, 0, 0)),
            pl.BlockSpec((1, N_PAD, HALF), lambda c: (c, 0, 0)),
            pl.BlockSpec((N_PAD, 16), lambda c: (0, 0)),
            pl.BlockSpec((N_PAD, 4), lambda c: (0, 0)),
            pl.BlockSpec((1, HEADS, HALF), lambda c: (c, 0, 0)),
            pl.BlockSpec((1, 1, HALF), lambda c: (c, 0, 0)),
            pl.BlockSpec((1, 1, HALF), lambda c: (c, 0, 0)),
            pl.BlockSpec((1, 1, HALF), lambda c: (c, 0, 0)),
        ],
        out_specs=pl.BlockSpec((1, N_PAD, HALF), lambda c: (c, 0, 0)),
        out_shape=jax.ShapeDtypeStruct((2, N_PAD, HALF), jnp.float32),
    )(msg, hb, den16, exl, S, b_gat.reshape(2, 1, HALF),
      bn_g.reshape(2, 1, HALF), bn_b.reshape(2, 1, HALF))

    sage_edge = pl.kernel(
        _sage_sc_body,
        compiler_params=sc_params,
        out_type=jax.ShapeDtypeStruct((2, N_PAD, HALF), jnp.float32),
        mesh=mesh,
        scratch_types=[
            pltpu.VMEM((SBS, CHS), jnp.int32),
            pltpu.VMEM((SBS, CHS), jnp.int32),
            pltpu.VMEM((4, CHS, HALF), jnp.float32),
            pltpu.VMEM_SHARED((N_PAD, HALF), jnp.float32),
            pltpu.SemaphoreType.DMA,
            pltpu.SemaphoreType.DMA,
            pltpu.SemaphoreType.DMA,
            pltpu.SemaphoreType.DMA,
            pltpu.SemaphoreType.DMA,
            pltpu.SemaphoreType.DMA,
            pltpu.SemaphoreType.DMA,
            pltpu.SemaphoreType.DMA,
        ],
    )
    src2s = src_p.reshape(NSUB * NCHS, CHS)
    dst2s = dst_p.reshape(NSUB * NCHS, CHS)
    agg = sage_edge(src2s, dst2s, h2, z128)

    B3 = N_PAD // 4
    out = pl.pallas_call(
        _tc3_body,
        grid=(4,),
        in_specs=[
            pl.BlockSpec((2, B3, HALF), lambda i: (0, i, 0)),
            pl.BlockSpec((2, B3, HALF), lambda i: (0, i, 0)),
            pl.BlockSpec((B3, 16), lambda i: (i, 0)),
            pl.BlockSpec((B3, IN_CH), lambda i: (i, 0)),
            pl.BlockSpec((HH, OUT_CH), lambda i: (0, 0)),
            pl.BlockSpec((HH, OUT_CH), lambda i: (0, 0)),
            pl.BlockSpec((IN_CH, OUT_CH), lambda i: (0, 0)),
            pl.BlockSpec((1, OUT_CH), lambda i: (0, 0)),
        ],
        out_specs=pl.BlockSpec((B3, OUT_CH), lambda i: (i, 0)),
        out_shape=jax.ShapeDtypeStruct((N_PAD, OUT_CH), jnp.float32),
    )(agg, h2, den16, x_p, W_sl.T, W_sr.T, W_skip.T,
      (b_s + b_skip).reshape(1, OUT_CH))

    return out[:N]


# speedup vs baseline: 1.1919x; 1.1919x over previous
import functools

import jax
import jax.numpy as jnp
from jax import lax
from jax.experimental import pallas as pl
from jax.experimental.pallas import tpu as pltpu
from jax.experimental.pallas import tpu_sc as plsc

N = 10000
E = 320000
IN_CH = 128
HID = 64
HEADS = 4
OUT_CH = 128
HH = HEADS * HID
HALF = HH // 2

NSUB = 16
CH = 80
N_PAD = 10112
NPS = N_PAD // NSUB
E_PAD = 327680
EPS = E_PAD // NSUB
NCH = EPS // CH
SB = 32
NSB = NCH // SB


def _tc1_body(x_ref, wgT_ref, ab_ref, h_ref, asrc_ref, adst_ref, exl_ref):
    x = x_ref[...]
    h = jnp.dot(x, wgT_ref[...], preferred_element_type=jnp.float32)
    h_ref[0] = h[:, :HALF]
    h_ref[1] = h[:, HALF:]
    a8 = jnp.dot(h, ab_ref[...], preferred_element_type=jnp.float32)
    z = jnp.zeros((x.shape[0], 12), jnp.float32)
    asrc_ref[...] = jnp.concatenate([a8[:, 0:4], z], axis=1)
    adst_ref[...] = jnp.concatenate([a8[:, 4:8], z], axis=1)
    al = a8[:, 0:4] + a8[:, 4:8]
    al = jnp.where(al > 0.0, al, 0.2 * al)
    exl_ref[...] = jnp.exp(al)


NPAIR = SB // 2
NQUAD = SB // 4


def _gat_sc_body(src2_h, dst2_h, asrc_h, adst_h, hsc_h, z128_h, z16_h,
                 msg_h, den_h,
                 sball, dball, ag, adg, exb, hrows, msg_sp, den_sp,
                 gsem0, gsem1, ssem0, ssem1):
    c = lax.axis_index("c")
    s = lax.axis_index("s")
    nbase = s * NPS
    pltpu.sync_copy(z128_h.at[pl.ds(nbase, NPS)], msg_sp.at[pl.ds(nbase, NPS)])
    pltpu.sync_copy(z16_h.at[pl.ds(nbase, NPS)], den_sp.at[pl.ds(nbase, NPS)])
    plsc.subcore_barrier()

    def run_half(h_half, msg_half, col0, do_den):
        gs = (gsem0, gsem1)
        ss = (ssem0, ssem1)

        def issue_gather(k, b):
            pltpu.async_copy(asrc_h.at[sball.at[k]], ag.at[b], gs[b])
            pltpu.async_copy(adst_h.at[dball.at[k]], adg.at[b], gs[b])
            pltpu.async_copy(h_half.at[sball.at[k]], hrows.at[b], gs[b])

        def wait_gather(k, b):
            pltpu.make_async_copy(asrc_h.at[sball.at[k]], ag.at[b], gs[b]).wait()
            pltpu.make_async_copy(adst_h.at[dball.at[k]], adg.at[b], gs[b]).wait()
            pltpu.make_async_copy(h_half.at[sball.at[k]], hrows.at[b], gs[b]).wait()

        def drain_scatter(k, b):
            pltpu.make_async_copy(hrows.at[b], msg_sp.at[dball.at[k]], ss[b]).wait()
            if do_den:
                pltpu.make_async_copy(exb.at[b], den_sp.at[dball.at[k]], ss[b]).wait()

        def compute_and_scatter(k, b):
            @plsc.parallel_loop(0, CH, step=1, unroll=4)
            def ebody(j):
                a = ag[b, j, :] + adg[b, j, :]
                a = jnp.maximum(a, a * 0.2)
                e = jnp.exp(a)
                exb[b, j, :] = e
                e0 = e[col0]
                e1 = e[col0 + 1]
                for v in range(4):
                    hrows[b, j, pl.ds(v * 16, 16)] = hrows[b, j, pl.ds(v * 16, 16)] * e0
                for v in range(4, 8):
                    hrows[b, j, pl.ds(v * 16, 16)] = hrows[b, j, pl.ds(v * 16, 16)] * e1
            pltpu.async_copy(hrows.at[b], msg_sp.at[dball.at[k]], ss[b], add=True)
            if do_den:
                pltpu.async_copy(exb.at[b], den_sp.at[dball.at[k]], ss[b], add=True)

        def super_t(t, carry):
            tb = s * NCH + t * SB
            pltpu.sync_copy(src2_h.at[pl.ds(tb, SB)], sball)
            pltpu.sync_copy(dst2_h.at[pl.ds(tb, SB)], dball)
            issue_gather(0, 0)

            def pairbody(i, carry2):
                k0 = 2 * i
                wait_gather(k0, 0)

                @pl.when(i > 0)
                def _():
                    drain_scatter(k0 - 1, 1)

                issue_gather(k0 + 1, 1)
                compute_and_scatter(k0, 0)

                wait_gather(k0 + 1, 1)
                drain_scatter(k0, 0)

                @pl.when(i < NPAIR - 1)
                def _():
                    issue_gather(k0 + 2, 0)

                compute_and_scatter(k0 + 1, 1)
                return carry2

            lax.fori_loop(0, NPAIR, pairbody, 0)
            drain_scatter(SB - 1, 1)
            return carry

        lax.fori_loop(0, NSB, super_t, 0)
        plsc.subcore_barrier()
        pltpu.sync_copy(msg_sp.at[pl.ds(nbase, NPS)], msg_half.at[pl.ds(nbase, NPS)])
        if do_den:
            pltpu.sync_copy(den_sp.at[pl.ds(nbase, NPS)], den_h.at[pl.ds(nbase, NPS)])

    @pl.when(c == 0)
    def _():
        run_half(hsc_h.at[0], msg_h.at[0], 0, True)

    @pl.when(c == 1)
    def _():
        run_half(hsc_h.at[1], msg_h.at[1], 2, False)


def _sage_sc_body(src2_h, dst2_h, h2_h, z128_h, agg_h,
                  sball, dball, hrows,
                  agg_sp,
                  gsem0, gsem1, gsem2, gsem3, ssem0, ssem1, ssem2, ssem3):
    c = lax.axis_index("c")
    s = lax.axis_index("s")
    nbase = s * NPS
    pltpu.sync_copy(z128_h.at[pl.ds(nbase, NPS)], agg_sp.at[pl.ds(nbase, NPS)])
    plsc.subcore_barrier()

    def run_half(h_half, out_half):
        gs = (gsem0, gsem1, gsem2, gsem3)
        ss = (ssem0, ssem1, ssem2, ssem3)

        def issue_gather(k, b):
            pltpu.async_copy(h_half.at[sball.at[k]], hrows.at[b], gs[b])

        def wait_gather(k, b):
            pltpu.make_async_copy(h_half.at[sball.at[k]], hrows.at[b], gs[b]).wait()

        def drain_scatter(k, b):
            pltpu.make_async_copy(hrows.at[b], agg_sp.at[dball.at[k]], ss[b]).wait()

        def super_t(t, carry):
            tb = s * NCH + t * SB
            pltpu.sync_copy(src2_h.at[pl.ds(tb, SB)], sball)
            pltpu.sync_copy(dst2_h.at[pl.ds(tb, SB)], dball)
            for b in range(3):
                issue_gather(b, b)

            def quadbody(i, carry2):
                for b in range(4):
                    k = 4 * i + b
                    bn = (b + 3) % 4
                    wait_gather(k, b)
                    if b == 0:
                        @pl.when(i > 0)
                        def _():
                            drain_scatter(k - 1, bn)

                        issue_gather(k + 3, bn)
                    else:
                        @pl.when(i < NQUAD - 1)
                        def _():
                            drain_scatter(k - 1, bn)
                            issue_gather(k + 3, bn)
                    pltpu.async_copy(hrows.at[b], agg_sp.at[dball.at[k]], ss[b], add=True)
                return carry2

            lax.fori_loop(0, NQUAD, quadbody, 0)
            for b in range(4):
                drain_scatter(SB - 4 + b, b)
            return carry

        lax.fori_loop(0, NSB, super_t, 0)
        plsc.subcore_barrier()
        pltpu.sync_copy(agg_sp.at[pl.ds(nbase, NPS)], out_half.at[pl.ds(nbase, NPS)])

    @pl.when(c == 0)
    def _():
        run_half(h2_h.at[0], agg_h.at[0])

    @pl.when(c == 1)
    def _():
        run_half(h2_h.at[1], agg_h.at[1])


def _tc2_body(msg_ref, h_ref, den_ref, exl_ref, S_ref, bg_ref, bng_ref, bnb_ref,
              h2_ref):
    exl = exl_ref[0:N, :]
    S = S_ref[0]
    elb = jnp.dot(exl, S, preferred_element_type=jnp.float32)
    r4 = 1.0 / (den_ref[0:N, 0:4] + exl + 1e-16)
    rb = jnp.dot(r4, S, preferred_element_type=jnp.float32)
    gat = (msg_ref[0, 0:N, :] + elb * h_ref[0, 0:N, :]) * rb + bg_ref[0]
    mu = jnp.mean(gat, axis=0, keepdims=True)
    xc = gat - mu
    var = jnp.mean(xc * xc, axis=0, keepdims=True)
    xn = xc * lax.rsqrt(var + 1e-5) * bng_ref[0] + bnb_ref[0]
    h2 = jnp.where(xn > 0.0, xn, jnp.exp(jnp.minimum(xn, 0.0)) - 1.0)
    h2_ref[0, 0:N, :] = h2
    h2_ref[0, N:N_PAD, :] = jnp.zeros((N_PAD - N, HALF), jnp.float32)


def _tc3_body(agg_ref, h2_ref, den_ref, x_ref, wsl_ref, wsr_ref, wsk_ref,
              bt_ref, out_ref):
    inv = 1.0 / jnp.maximum(den_ref[:, 4:5], 1.0)
    m0 = agg_ref[0] * inv
    m1 = agg_ref[1] * inv
    out = jnp.dot(m0, wsl_ref[0:HALF, :], preferred_element_type=jnp.float32)
    out = out + jnp.dot(m1, wsl_ref[HALF:HH, :], preferred_element_type=jnp.float32)
    out = out + jnp.dot(h2_ref[0], wsr_ref[0:HALF, :], preferred_element_type=jnp.float32)
    out = out + jnp.dot(h2_ref[1], wsr_ref[HALF:HH, :], preferred_element_type=jnp.float32)
    out = out + jnp.dot(x_ref[...], wsk_ref[...], preferred_element_type=jnp.float32)
    out_ref[...] = out + bt_ref[...]


def kernel(x, edge_index, W_gat, att_src, att_dst, b_gat, bn_g, bn_b,
           W_sl, W_sr, b_s, W_skip, b_skip):
    src = edge_index[0]
    dst = edge_index[1]
    pad_idx = (jnp.arange(E_PAD - E, dtype=jnp.int32) % 16) + N
    src_p = jnp.concatenate([src, pad_idx])
    dst_p = jnp.concatenate([dst, pad_idx])
    x_p = jnp.pad(x, ((0, N_PAD - N), (0, 0)))

    eye = jnp.eye(HEADS, dtype=jnp.float32)
    A_src = (att_src[:, :, None] * eye[:, None, :]).reshape(HH, HEADS)
    A_dst = (att_dst[:, :, None] * eye[:, None, :]).reshape(HH, HEADS)
    AB = jnp.concatenate([A_src, A_dst], axis=1)

    fhead = jnp.arange(HALF, dtype=jnp.int32) // HID
    g = jnp.arange(HEADS, dtype=jnp.int32)
    S = (g[None, :, None] == (2 * jnp.arange(2, dtype=jnp.int32)[:, None, None]
                              + fhead[None, None, :])).astype(jnp.float32)

    z128 = jnp.zeros((N_PAD, HALF), jnp.float32)
    z16 = jnp.zeros((N_PAD, 16), jnp.float32)

    B1 = N_PAD // 4
    h_sc, asrc_p, adst_p, exl = pl.pallas_call(
        _tc1_body,
        grid=(4,),
        in_specs=[
            pl.BlockSpec((B1, IN_CH), lambda i: (i, 0)),
            pl.BlockSpec((IN_CH, HH), lambda i: (0, 0)),
            pl.BlockSpec((HH, 8), lambda i: (0, 0)),
        ],
        out_specs=[
            pl.BlockSpec((2, B1, HALF), lambda i: (0, i, 0)),
            pl.BlockSpec((B1, 16), lambda i: (i, 0)),
            pl.BlockSpec((B1, 16), lambda i: (i, 0)),
            pl.BlockSpec((B1, 4), lambda i: (i, 0)),
        ],
        out_shape=[
            jax.ShapeDtypeStruct((2, N_PAD, HALF), jnp.float32),
            jax.ShapeDtypeStruct((N_PAD, 16), jnp.float32),
            jax.ShapeDtypeStruct((N_PAD, 16), jnp.float32),
            jax.ShapeDtypeStruct((N_PAD, 4), jnp.float32),
        ],
    )(x_p, W_gat.T, AB)

    mesh = plsc.VectorSubcoreMesh(core_axis_name="c", subcore_axis_name="s")

    sc_params = pltpu.CompilerParams(use_tc_tiling_on_sc=False)

    gat_edge = pl.kernel(
        _gat_sc_body,
        compiler_params=sc_params,
        out_type=[
            jax.ShapeDtypeStruct((2, N_PAD, HALF), jnp.float32),
            jax.ShapeDtypeStruct((N_PAD, 16), jnp.float32),
        ],
        mesh=mesh,
        scratch_types=[
            pltpu.VMEM((SB, CH), jnp.int32),
            pltpu.VMEM((SB, CH), jnp.int32),
            pltpu.VMEM((2, CH, 16), jnp.float32),
            pltpu.VMEM((2, CH, 16), jnp.float32),
            pltpu.VMEM((2, CH, 16), jnp.float32),
            pltpu.VMEM((2, CH, HALF), jnp.float32),
            pltpu.VMEM_SHARED((N_PAD, HALF), jnp.float32),
            pltpu.VMEM_SHARED((N_PAD, 16), jnp.float32),
            pltpu.SemaphoreType.DMA,
            pltpu.SemaphoreType.DMA,
            pltpu.SemaphoreType.DMA,
            pltpu.SemaphoreType.DMA,
        ],
    )
    src2 = src_p.reshape(NSUB * NCH, CH)
    dst2 = dst_p.reshape(NSUB * NCH, CH)
    msg, den16 = gat_edge(src2, dst2, asrc_p, adst_p, h_sc, z128, z16)

    h2 = pl.pallas_call(
        _tc2_body,
        grid=(2,),
        in_specs=[
            pl.BlockSpec((1, N_PAD, HALF), lambda c: (c, 0, 0)),
            pl.BlockSpec((1, N_PAD, HALF), lambda c: (c, 0, 0)),
            pl.BlockSpec((N_PAD, 16), lambda c: (0, 0)),
            pl.BlockSpec((N_PAD, 4), lambda c: (0, 0)),
            pl.BlockSpec((1, HEADS, HALF), lambda c: (c, 0, 0)),
            pl.BlockSpec((1, 1, HALF), lambda c: (c, 0, 0)),
            pl.BlockSpec((1, 1, HALF), lambda c: (c, 0, 0)),
            pl.BlockSpec((1, 1, HALF), lambda c: (c, 0, 0)),
        ],
        out_specs=pl.BlockSpec((1, N_PAD, HALF), lambda c: (c, 0, 0)),
        out_shape=jax.ShapeDtypeStruct((2, N_PAD, HALF), jnp.float32),
    )(msg, h_sc, den16, exl, S, b_gat.reshape(2, 1, HALF),
      bn_g.reshape(2, 1, HALF), bn_b.reshape(2, 1, HALF))

    sage_edge = pl.kernel(
        _sage_sc_body,
        compiler_params=sc_params,
        out_type=jax.ShapeDtypeStruct((2, N_PAD, HALF), jnp.float32),
        mesh=mesh,
        scratch_types=[
            pltpu.VMEM((SB, CH), jnp.int32),
            pltpu.VMEM((SB, CH), jnp.int32),
            pltpu.VMEM((4, CH, HALF), jnp.float32),
            pltpu.VMEM_SHARED((N_PAD, HALF), jnp.float32),
            pltpu.SemaphoreType.DMA,
            pltpu.SemaphoreType.DMA,
            pltpu.SemaphoreType.DMA,
            pltpu.SemaphoreType.DMA,
            pltpu.SemaphoreType.DMA,
            pltpu.SemaphoreType.DMA,
            pltpu.SemaphoreType.DMA,
            pltpu.SemaphoreType.DMA,
        ],
    )
    agg = sage_edge(src2, dst2, h2, z128)

    B3 = N_PAD // 4
    out = pl.pallas_call(
        _tc3_body,
        grid=(4,),
        in_specs=[
            pl.BlockSpec((2, B3, HALF), lambda i: (0, i, 0)),
            pl.BlockSpec((2, B3, HALF), lambda i: (0, i, 0)),
            pl.BlockSpec((B3, 16), lambda i: (i, 0)),
            pl.BlockSpec((B3, IN_CH), lambda i: (i, 0)),
            pl.BlockSpec((HH, OUT_CH), lambda i: (0, 0)),
            pl.BlockSpec((HH, OUT_CH), lambda i: (0, 0)),
            pl.BlockSpec((IN_CH, OUT_CH), lambda i: (0, 0)),
            pl.BlockSpec((1, OUT_CH), lambda i: (0, 0)),
        ],
        out_specs=pl.BlockSpec((B3, OUT_CH), lambda i: (i, 0)),
        out_shape=jax.ShapeDtypeStruct((N_PAD, OUT_CH), jnp.float32),
    )(agg, h2, den16, x_p, W_sl.T, W_sr.T, W_skip.T,
      (b_s + b_skip).reshape(1, OUT_CH))

    return out[:N]
